# baseline (speedup 1.0000x reference)
import jax
import jax.numpy as jnp
from jax.experimental import pallas as pl

CUTOFF = 5.0
P = 5
NS = 7
NR = 6
NBLK = 4
N = 10000
E = 160000


def _envelope(x):
    a = -(P + 1) * (P + 2) / 2.0
    b = P * (P + 2)
    c = -P * (P + 1) / 2.0
    return 1.0 / x + a * x ** (P - 1) + b * x ** P + c * x ** (P + 1)


def _copy_kernel(x_ref, o_ref):
    o_ref[...] = x_ref[...]


def kernel(Z, R, edge_index, l_edge_index, rbf_freq, sbf_freq, emb_z, W_emb, b_emb,
           ib_Wji, ib_Wkj, ib_rbf_W, ib_sbf_W, ib_bilin, ib_res1, ib_skip, ib_res2,
           ib_res3, ob_rbf_W, ob_dense, ob_out):
    src = edge_index[0]
    dst = edge_index[1]
    l_src = l_edge_index[0]
    l_dst = l_edge_index[1]
    o = R[dst] - R[src]
    d = jnp.sqrt(jnp.sum(o * o, axis=-1) + 1e-12)
    xs = d / CUTOFF
    env = _envelope(xs)
    rbf = env[:, None] * jnp.sin(rbf_freq[None, :] * xs[:, None])
    rbf_env = env[:, None] * jnp.sin(sbf_freq[None, :] * xs[:, None])
    h_z = emb_z[Z]
    m = jax.nn.silu(jnp.concatenate([h_z[src], h_z[dst], rbf], axis=-1) @ W_emb + b_emb)
    o1 = o[l_src]
    o2 = o[l_dst]
    dotp = jnp.sum(o1 * o2, axis=-1)
    crs = jnp.cross(o1, o2)
    crn = jnp.sqrt(jnp.sum(crs * crs, axis=-1) + 1e-12)
    angle = jnp.arctan2(crn, dotp)
    cbf = jnp.stack([jnp.cos(l * angle) for l in range(NS)], axis=1)
    cbf = jnp.repeat(cbf, NR, axis=1)
    sbf = rbf_env[l_src] * cbf

    def output_block(b, mm):
        t = (rbf @ ob_rbf_W[b]) * mm
        node = jax.ops.segment_sum(t, dst, num_segments=N)
        for j in range(3):
            node = jax.nn.silu(node @ ob_dense[b, j])
        return node @ ob_out[b]

    Pout = output_block(0, m)
    for i in range(NBLK):
        x_ji = jax.nn.silu(m @ ib_Wji[i])
        x_kj = jax.nn.silu(m @ ib_Wkj[i])
        x_kj = x_kj * (rbf @ ib_rbf_W[i])
        xk = x_kj[l_src]
        sb = sbf @ ib_sbf_W[i]
        bil = jnp.einsum('wj,wl,ijl->wi', sb, xk, ib_bilin[i])
        agg = jax.ops.segment_sum(bil, l_dst, num_segments=E)
        h = x_ji + agg
        h = h + jax.nn.silu(h @ ib_res1[i])
        h = jax.nn.silu(h @ ib_skip[i]) + m
        h = h + jax.nn.silu(h @ ib_res2[i])
        h = h + jax.nn.silu(h @ ib_res3[i])
        m = h
        Pout = Pout + output_block(i + 1, m)

    Pout = pl.pallas_call(
        _copy_kernel,
        out_shape=jax.ShapeDtypeStruct(Pout.shape, Pout.dtype),
    )(Pout)
    return Pout


# speedup vs baseline: 1.1059x; 1.1059x over previous
import functools

import jax
import jax.numpy as jnp
from jax import lax
from jax.experimental import pallas as pl
from jax.experimental.pallas import tpu as pltpu
from jax.experimental.pallas import tpu_sc as plsc

N = 10000
E = 160000
E2 = 320000
EMB = 64
NR = 6
NS = 7
NSR = NS * NR
NB = 8
NBLK = 4
NT = 12
CUTOFF = 5.0
P = 5

NC = 2
NSUB = 16
NWORK = NC * NSUB
LANES = 16

N_PAD = 10240
HZ_PAD = 32768
E_PAD = 163840
E2_PAD = 327680

RB = 640

_MESH = dict(core_axis_name="c", subcore_axis_name="s", num_cores=NC,
             num_subcores=NSUB)


def _silu(x):
    return x / (1.0 + jnp.exp(-x))


def _make_gather(V, D, B_pad, G, NGRP):
    bt = B_pad // NWORK
    assert bt == G * NGRP * 128

    def body(table, idx2, out, ichunk, rows, gsem, osem):
        c = lax.axis_index("c")
        s = lax.axis_index("s")
        wid = s * NC + c
        base = wid * bt

        def group(g, _):
            row0 = pl.multiple_of((base + g * (G * 128)) // 128, 8)
            pltpu.sync_copy(idx2.at[pl.ds(row0, G)], ichunk)
            cps = []
            for b in range(G):
                cps.append(pltpu.async_copy(
                    table.at[ichunk.at[b]], rows.at[b], gsem))
            for cp in cps:
                cp.wait()
            ops = []
            for b in range(G):
                off = pl.multiple_of(base + g * (G * 128) + b * 128, 128)
                ops.append(pltpu.async_copy(
                    rows.at[b], out.at[pl.ds(off, 128)], osem))
            for cp in ops:
                cp.wait()
            return 0

        lax.fori_loop(0, NGRP, group, 0)

    fn = pl.kernel(
        body,
        out_type=jax.ShapeDtypeStruct((B_pad, D), jnp.float32),
        mesh=plsc.VectorSubcoreMesh(**_MESH),
        compiler_params=pltpu.CompilerParams(use_tc_tiling_on_sc=False),
        scratch_types=[
            pltpu.VMEM((G, 128), jnp.int32),
            pltpu.VMEM((G, 128, D), jnp.float32),
            pltpu.SemaphoreType.DMA,
            pltpu.SemaphoreType.DMA,
        ],
    )
    return fn


def _make_seg16(W, W_pad, G, NGRP, rounds, dc, r_acc, stripe, subch, out_rows):
    wt = W_pad // NSUB
    assert wt == G * NGRP * 128
    assert stripe * NSUB == r_acc
    zrows = stripe // 4 if stripe % 4 == 0 else stripe
    nz = stripe // zrows
    trash = r_acc - 8

    def body(vals3, idx2, out, ichunk, didx, rows, zbuf, acc, gsem, ssem):
        c = lax.axis_index("c")
        s = lax.axis_index("s")

        zv = jnp.zeros((LANES,), jnp.float32)

        def zb(i, _):
            zbuf[i, :] = zv
            return 0
        lax.fori_loop(0, zrows, zb, 0)

        for (fb0, lo0, fb1, lo1) in rounds:
            fb = jnp.where(c == 0, fb0, fb1)
            lo = jnp.where(c == 0, lo0, lo1)
            for z in range(nz):
                pltpu.sync_copy(zbuf, acc.at[pl.ds(s * stripe + z * zrows,
                                                   zrows)])
            plsc.subcore_barrier()
            base = s * wt

            def group(g, _):
                row0 = pl.multiple_of((base + g * (G * 128)) // 128, 8)
                pltpu.sync_copy(idx2.at[pl.ds(row0, G)], ichunk)
                for b in range(G):
                    for k in range(8):
                        iv = ichunk[b, pl.ds(k * 16, 16)]
                        pos = (base + g * (G * 128) + b * 128 + k * 16
                               + lax.iota(jnp.int32, 16))
                        ok = ((pos < W) & (iv >= lo) & (iv < lo + dc))
                        didx[b, pl.ds(k * 16, 16)] = jnp.where(
                            ok, iv - lo, trash)
                cps = []
                for b in range(G):
                    w0 = pl.multiple_of(base + g * (G * 128) + b * 128, 128)
                    cps.append(pltpu.async_copy(
                        vals3.at[pl.ds(w0, 128), fb], rows.at[b], gsem))
                for cp in cps:
                    cp.wait()
                ops = []
                for b in range(G):
                    ops.append(pltpu.async_copy(
                        rows.at[b], acc.at[didx.at[b]], ssem, add=True))
                for cp in ops:
                    cp.wait()
                return 0

            lax.fori_loop(0, NGRP, group, 0)
            plsc.subcore_barrier()
            for sc_i in range(stripe // subch):
                start = s * subch * (stripe // subch) + sc_i * subch

                @pl.when(start < dc)
                def _():
                    st = pl.multiple_of(start, subch)
                    dst0 = pl.multiple_of(lo + st, 8)
                    pltpu.sync_copy(
                        acc.at[pl.ds(st, subch)],
                        out.at[pl.ds(dst0, subch), fb])
            plsc.subcore_barrier()

    fn = pl.kernel(
        body,
        out_type=jax.ShapeDtypeStruct((out_rows, 4, 16), jnp.float32),
        mesh=plsc.VectorSubcoreMesh(**_MESH),
        compiler_params=pltpu.CompilerParams(use_tc_tiling_on_sc=False),
        scratch_types=[
            pltpu.VMEM((G, 128), jnp.int32),
            pltpu.VMEM((G, 128), jnp.int32),
            pltpu.VMEM((G, 128, 16), jnp.float32),
            pltpu.VMEM((zrows, 16), jnp.float32),
            pltpu.VMEM_SHARED((r_acc, 16), jnp.float32),
            pltpu.SemaphoreType.DMA,
            pltpu.SemaphoreType.DMA,
        ],
    )
    return fn


_seg_edge = None
_seg_node = None


def _get_seg_edge():
    global _seg_edge
    if _seg_edge is None:
        _seg_edge = _make_seg16(
            W=E2, W_pad=E2_PAD, G=8, NGRP=20,
            rounds=[(r, 0, r, 80000) for r in range(4)],
            dc=80000, r_acc=81920, stripe=5120, subch=320, out_rows=E_PAD)
    return _seg_edge


def _get_seg_node():
    global _seg_node
    if _seg_node is None:
        _seg_node = _make_seg16(
            W=E, W_pad=E_PAD, G=8, NGRP=10,
            rounds=[(0, 0, 2, 0), (1, 0, 3, 0)],
            dc=N_PAD, r_acc=N_PAD, stripe=640, subch=640, out_rows=N_PAD)
    return _seg_node


_gathers = {}


def _get_gather(V, D, B_pad, G, NGRP):
    key = (V, D, B_pad, G, NGRP)
    if key not in _gathers:
        _gathers[key] = _make_gather(V, D, B_pad, G, NGRP)
    return _gathers[key]


def _row_spec(d):
    return pl.BlockSpec((RB, d), lambda i: (i, 0))


def _full_spec(shape):
    nd = len(shape)
    return pl.BlockSpec(shape, lambda i, _n=nd: (0,) * _n)


def _tc_call(body, nblocks, in_specs, out_specs, out_shapes):
    return pl.pallas_call(
        body,
        grid=(nblocks,),
        in_specs=in_specs,
        out_specs=out_specs,
        out_shape=out_shapes,
    )


def _prep_body(hz_ref, rp_ref, w1_ref, w2_ref, at_ref, bt_ref):
    hz = hz_ref[...]
    rp = rp_ref[...]
    z8 = jnp.zeros((RB, 8), jnp.float32)
    a = jnp.dot(hz, w1_ref[...], preferred_element_type=jnp.float32)
    b = jnp.dot(hz, w2_ref[...], preferred_element_type=jnp.float32)
    at_ref[...] = jnp.concatenate([a, rp, z8], axis=1)
    bt_ref[...] = jnp.concatenate([b, rp, z8], axis=1)


def _edge_body(ag_ref, bg_ref, w3_ref, bias_ref, obw0_ref,
               m_ref, ren_ref, o16_ref, rbf8_ref, t0_ref):
    ag = ag_ref[...]
    bg = bg_ref[...]
    ha = ag[:, 0:64]
    hb = bg[:, 0:64]
    o = bg[:, 64:67] - ag[:, 64:67]
    d2 = jnp.sum(o * o, axis=1, keepdims=True) + 1e-12
    d = jnp.sqrt(d2)
    xs = d * (1.0 / CUTOFF)
    inv = 1.0 / xs
    a_c = -(P + 1) * (P + 2) / 2.0
    b_c = float(P * (P + 2))
    c_c = -P * (P + 1) / 2.0
    x4 = (xs * xs) * (xs * xs)
    env = inv + a_c * x4 + b_c * x4 * xs + c_c * x4 * xs * xs
    th = jnp.float32(jnp.pi) * xs
    s_arr = jnp.sin(th)
    c_arr = jnp.cos(th)
    for w in (1, 2, 4, 8, 16):
        sw = s_arr[:, w - 1:w]
        cw = c_arr[:, w - 1:w]
        s_new = jnp.concatenate([s_arr, s_arr * cw + c_arr * sw], axis=1)
        c_new = jnp.concatenate([c_arr, c_arr * cw - s_arr * sw], axis=1)
        s_arr, c_arr = s_new, c_new
    s32 = s_arr[:, 31:32]
    c32 = c_arr[:, 31:32]
    s48 = jnp.concatenate(
        [s_arr, s_arr[:, 0:16] * c32 + c_arr[:, 0:16] * s32], axis=1)
    renv = env * s48
    z13 = jnp.zeros((RB, 13), jnp.float32)
    ren_ref[...] = jnp.concatenate([renv, o, z13], axis=1)
    o16_ref[...] = jnp.concatenate([o, z13], axis=1)
    rbf8 = jnp.concatenate(
        [renv[:, 0:6], jnp.zeros((RB, 2), jnp.float32)], axis=1)
    rbf8_ref[...] = rbf8
    pre = (ha + hb + jnp.dot(rbf8, w3_ref[...],
                             preferred_element_type=jnp.float32)
           + bias_ref[0:1, :])
    m = _silu(pre)
    m_ref[...] = m
    t0_ref[...] = jnp.dot(rbf8, obw0_ref[...],
                          preferred_element_type=jnp.float32) * m


def _line_body(reng_ref, o16g_ref, wcat_ref, sb_ref):
    reng = reng_ref[...]
    re1 = reng[:, 0:48]
    o1 = reng[:, 48:51]
    o2 = o16g_ref[...][:, 0:3]
    dotp = jnp.sum(o1 * o2, axis=1, keepdims=True)
    cx = o1[:, 1:2] * o2[:, 2:3] - o1[:, 2:3] * o2[:, 1:2]
    cy = o1[:, 2:3] * o2[:, 0:1] - o1[:, 0:1] * o2[:, 2:3]
    cz = o1[:, 0:1] * o2[:, 1:2] - o1[:, 1:2] * o2[:, 0:1]
    crn2 = cx * cx + cy * cy + cz * cz + 1e-12
    hyp = jnp.sqrt(dotp * dotp + crn2)
    ca = dotp / hyp
    ts = [jnp.ones((RB, 1), jnp.float32), ca]
    for _ in range(2, NS):
        ts.append(2.0 * ca * ts[-1] - ts[-2])
    parts = [jnp.broadcast_to(t, (RB, NR)) for t in ts]
    parts.append(jnp.zeros((RB, NR), jnp.float32))
    cbf = jnp.concatenate(parts, axis=1)
    sb_ref[...] = jnp.dot(re1 * cbf, wcat_ref[...],
                          preferred_element_type=jnp.float32)


def _c1_body(m_ref, rbf8_ref, wji_ref, wkj_ref, rbfw_ref,
             xji_ref, xkj_ref):
    m = m_ref[...]
    rbf8 = rbf8_ref[...]
    xji_ref[...] = _silu(jnp.dot(m, wji_ref[...],
                                 preferred_element_type=jnp.float32))
    xkj = _silu(jnp.dot(m, wkj_ref[...], preferred_element_type=jnp.float32))
    xkj_ref[...] = xkj * jnp.dot(rbf8, rbfw_ref[...],
                                 preferred_element_type=jnp.float32)


def _c2_body(xk_ref, sb_ref, mt_ref, bil_ref, *, blk):
    xk = xk_ref[...]
    sb = sb_ref[...]
    acc = jnp.zeros((RB, EMB), jnp.float32)
    for j in range(NB):
        acc = acc + jnp.dot(xk, mt_ref[j], preferred_element_type=jnp.float32
                            ) * sb[:, blk * 8 + j:blk * 8 + j + 1]
    bil_ref[...] = acc


def _c3_body(xji_ref, agg_ref, m_ref, rbf8_ref,
             r1_ref, sk_ref, r2_ref, r3_ref, obw_ref,
             mnew_ref, t_ref):
    h = xji_ref[...] + agg_ref[...]
    h = h + _silu(jnp.dot(h, r1_ref[...], preferred_element_type=jnp.float32))
    h = _silu(jnp.dot(h, sk_ref[...],
                      preferred_element_type=jnp.float32)) + m_ref[...]
    h = h + _silu(jnp.dot(h, r2_ref[...], preferred_element_type=jnp.float32))
    h = h + _silu(jnp.dot(h, r3_ref[...], preferred_element_type=jnp.float32))
    mnew_ref[...] = h
    t_ref[...] = jnp.dot(rbf8_ref[...], obw_ref[...],
                         preferred_element_type=jnp.float32) * h


def _out_body(n0_ref, n1_ref, n2_ref, n3_ref, n4_ref,
              dense_ref, outw_ref, p_ref):
    p = jnp.zeros((RB, 16), jnp.float32)
    nrefs = (n0_ref, n1_ref, n2_ref, n3_ref, n4_ref)
    for b in range(NBLK + 1):
        n = nrefs[b][...]
        for j in range(3):
            n = _silu(jnp.dot(n, dense_ref[b, j],
                              preferred_element_type=jnp.float32))
        p = p + jnp.dot(n, outw_ref[b], preferred_element_type=jnp.float32)
    p_ref[...] = p


def _pad_rows(x, rows):
    return jnp.pad(x, ((0, rows - x.shape[0]),) + ((0, 0),) * (x.ndim - 1))


def _pad_idx(ix, n):
    ix = ix.astype(jnp.int32)
    return jnp.pad(ix, (0, n - ix.shape[0])).reshape(-1, 128)


def kernel(Z, R, edge_index, l_edge_index, rbf_freq, sbf_freq, emb_z, W_emb,
           b_emb, ib_Wji, ib_Wkj, ib_rbf_W, ib_sbf_W, ib_bilin, ib_res1,
           ib_skip, ib_res2, ib_res3, ob_rbf_W, ob_dense, ob_out):
    f32 = jnp.float32
    src = edge_index[0]
    dst = edge_index[1]
    l_src = l_edge_index[0]
    l_dst = l_edge_index[1]

    zp = _pad_idx(Z, HZ_PAD)
    srcp = _pad_idx(src, E_PAD)
    dstp = _pad_idx(dst, E_PAD)
    l_srcp = _pad_idx(l_src, E2_PAD)
    l_dstp = _pad_idx(l_dst, E2_PAD)

    hz = _get_gather(95, EMB, HZ_PAD, 8, 1)(emb_z, zp)[:N_PAD]

    rp = _pad_rows(jnp.pad(R, ((0, 0), (0, 5))), N_PAD)
    w1 = W_emb[0:EMB]
    w2 = W_emb[EMB:2 * EMB]
    at, bt = _tc_call(
        _prep_body, N_PAD // RB,
        [_row_spec(EMB), _row_spec(8), _full_spec((EMB, EMB)),
         _full_spec((EMB, EMB))],
        [_row_spec(80), _row_spec(80)],
        [jax.ShapeDtypeStruct((N_PAD, 80), f32)] * 2,
    )(hz, rp, w1, w2)

    gat_e80 = _get_gather(N_PAD, 80, E_PAD, 8, 5)
    ag = gat_e80(at, srcp)
    bg = gat_e80(bt, dstp)

    w3p = jnp.pad(W_emb[2 * EMB:], ((0, 2), (0, 0)))
    biasp = jnp.broadcast_to(b_emb[None, :], (8, EMB))
    obwp = jnp.pad(ob_rbf_W, ((0, 0), (0, 2), (0, 0)))
    m, ren, o16, rbf8, t0 = _tc_call(
        _edge_body, E_PAD // RB,
        [_row_spec(80), _row_spec(80), _full_spec((8, EMB)),
         _full_spec((8, EMB)), _full_spec((8, EMB))],
        [_row_spec(EMB), _row_spec(EMB), _row_spec(16), _row_spec(8),
         _row_spec(EMB)],
        [jax.ShapeDtypeStruct((E_PAD, EMB), f32),
         jax.ShapeDtypeStruct((E_PAD, EMB), f32),
         jax.ShapeDtypeStruct((E_PAD, 16), f32),
         jax.ShapeDtypeStruct((E_PAD, 8), f32),
         jax.ShapeDtypeStruct((E_PAD, EMB), f32)],
    )(ag, bg, w3p, biasp, obwp[0])

    gat_l64 = _get_gather(E_PAD, EMB, E2_PAD, 8, 10)
    gat_l16 = _get_gather(E_PAD, 16, E2_PAD, 8, 10)
    reng = gat_l64(ren, l_srcp)
    o16g = gat_l16(o16, l_dstp)

    wcat = jnp.pad(
        jnp.transpose(ib_sbf_W, (1, 0, 2)).reshape(NSR, NBLK * NB),
        ((0, 48 - NSR), (0, 0)))
    (sb,) = _tc_call(
        _line_body, E2_PAD // RB,
        [_row_spec(EMB), _row_spec(16), _full_spec((48, NBLK * NB))],
        [_row_spec(NBLK * NB)],
        [jax.ShapeDtypeStruct((E2_PAD, NBLK * NB), f32)],
    )(reng, o16g, wcat)

    seg_edge = _get_seg_edge()
    seg_node = _get_seg_node()

    nodes = [seg_node(t0.reshape(E_PAD, 4, 16), dstp).reshape(N_PAD, EMB)]
    rbfwp = jnp.pad(ib_rbf_W, ((0, 0), (0, 2), (0, 0)))
    for i in range(NBLK):
        xji, xkj = _tc_call(
            _c1_body, E_PAD // RB,
            [_row_spec(EMB), _row_spec(8), _full_spec((EMB, EMB)),
             _full_spec((EMB, EMB)), _full_spec((8, EMB))],
            [_row_spec(EMB), _row_spec(EMB)],
            [jax.ShapeDtypeStruct((E_PAD, EMB), f32)] * 2,
        )(m, rbf8, ib_Wji[i], ib_Wkj[i], rbfwp[i])
        xkg = gat_l64(xkj, l_srcp)
        mt = jnp.transpose(ib_bilin[i], (1, 2, 0))
        (bil,) = _tc_call(
            functools.partial(_c2_body, blk=i), E2_PAD // RB,
            [_row_spec(EMB), _row_spec(NBLK * NB),
             _full_spec((NB, EMB, EMB))],
            [_row_spec(EMB)],
            [jax.ShapeDtypeStruct((E2_PAD, EMB), f32)],
        )(xkg, sb, mt)
        agg = seg_edge(bil.reshape(E2_PAD, 4, 16), l_dstp).reshape(E_PAD, EMB)
        m, t = _tc_call(
            _c3_body, E_PAD // RB,
            [_row_spec(EMB), _row_spec(EMB), _row_spec(EMB), _row_spec(8),
             _full_spec((EMB, EMB)), _full_spec((EMB, EMB)),
             _full_spec((EMB, EMB)), _full_spec((EMB, EMB)),
             _full_spec((8, EMB))],
            [_row_spec(EMB), _row_spec(EMB)],
            [jax.ShapeDtypeStruct((E_PAD, EMB), f32)] * 2,
        )(xji, agg, m, rbf8, ib_res1[i], ib_skip[i], ib_res2[i], ib_res3[i],
          obwp[i + 1])
        nodes.append(seg_node(t.reshape(E_PAD, 4, 16), dstp).reshape(N_PAD, EMB))

    outwp = jnp.pad(ob_out, ((0, 0), (0, 0), (0, 16 - NT)))
    (pout,) = _tc_call(
        _out_body, N_PAD // RB,
        [_row_spec(EMB)] * 5 + [_full_spec((NBLK + 1, 3, EMB, EMB)),
                                _full_spec((NBLK + 1, EMB, 16))],
        [_row_spec(16)],
        [jax.ShapeDtypeStruct((N_PAD, 16), f32)],
    )(*nodes, ob_dense, outwp)
    return pout[:N, :NT]


# speedup vs baseline: 1.1061x; 1.0002x over previous
import functools

import jax
import jax.numpy as jnp
from jax import lax
from jax.experimental import pallas as pl
from jax.experimental.pallas import tpu as pltpu
from jax.experimental.pallas import tpu_sc as plsc

N = 10000
E = 160000
E2 = 320000
EMB = 64
NR = 6
NS = 7
NSR = NS * NR
NB = 8
NBLK = 4
NT = 12
CUTOFF = 5.0
P = 5

NC = 2
NSUB = 16
NWORK = NC * NSUB
LANES = 16

N_PAD = 10240
HZ_PAD = 32768
E_PAD = 163840
E2_PAD = 327680

RB = 640

_MESH = dict(core_axis_name="c", subcore_axis_name="s", num_cores=NC,
             num_subcores=NSUB)


def _silu(x):
    return x / (1.0 + jnp.exp(-x))


def _make_gather(V, D, B_pad, G, NGRP, C=128):
    bt = B_pad // NWORK
    assert bt == G * NGRP * C

    def body(table, idx2, out, ichunk, rows, gsem, osem):
        c = lax.axis_index("c")
        s = lax.axis_index("s")
        wid = s * NC + c
        base = wid * bt

        def group(g, _):
            row0 = pl.multiple_of((base + g * (G * C)) // C, 8)
            pltpu.sync_copy(idx2.at[pl.ds(row0, G)], ichunk)
            cps = []
            for b in range(G):
                cps.append(pltpu.async_copy(
                    table.at[ichunk.at[b]], rows.at[b], gsem))
            for cp in cps:
                cp.wait()
            ops = []
            for b in range(G):
                off = pl.multiple_of(base + g * (G * C) + b * C, 16)
                ops.append(pltpu.async_copy(
                    rows.at[b], out.at[pl.ds(off, C)], osem))
            for cp in ops:
                cp.wait()
            return 0

        lax.fori_loop(0, NGRP, group, 0)

    fn = pl.kernel(
        body,
        out_type=jax.ShapeDtypeStruct((B_pad, D), jnp.float32),
        mesh=plsc.VectorSubcoreMesh(**_MESH),
        compiler_params=pltpu.CompilerParams(use_tc_tiling_on_sc=False),
        scratch_types=[
            pltpu.VMEM((G, C), jnp.int32),
            pltpu.VMEM((G, C, D), jnp.float32),
            pltpu.SemaphoreType.DMA,
            pltpu.SemaphoreType.DMA,
        ],
    )
    return fn


def _make_seg16(W, W_pad, G, NGRP, rounds, dc, r_acc, stripe, subch, out_rows,
                C=128):
    wt = W_pad // NSUB
    assert wt == G * NGRP * C
    assert stripe * NSUB == r_acc
    zrows = stripe // 4 if stripe % 4 == 0 else stripe
    nz = stripe // zrows
    trash = r_acc - 8

    def body(vals3, idx2, out, ichunk, didx, rows, zbuf, acc, gsem, ssem):
        c = lax.axis_index("c")
        s = lax.axis_index("s")

        zv = jnp.zeros((LANES,), jnp.float32)

        def zb(i, _):
            zbuf[i, :] = zv
            return 0
        lax.fori_loop(0, zrows, zb, 0)

        for (fb0, lo0, fb1, lo1) in rounds:
            fb = jnp.where(c == 0, fb0, fb1)
            lo = jnp.where(c == 0, lo0, lo1)
            for z in range(nz):
                pltpu.sync_copy(zbuf, acc.at[pl.ds(s * stripe + z * zrows,
                                                   zrows)])
            plsc.subcore_barrier()
            base = s * wt

            def group(g, _):
                row0 = pl.multiple_of((base + g * (G * C)) // C, 8)
                pltpu.sync_copy(idx2.at[pl.ds(row0, G)], ichunk)
                for b in range(G):
                    for k in range(C // 16):
                        iv = ichunk[b, pl.ds(k * 16, 16)]
                        pos = (base + g * (G * C) + b * C + k * 16
                               + lax.iota(jnp.int32, 16))
                        ok = ((pos < W) & (iv >= lo) & (iv < lo + dc))
                        didx[b, pl.ds(k * 16, 16)] = jnp.where(
                            ok, iv - lo, trash)
                cps = []
                for b in range(G):
                    w0 = pl.multiple_of(base + g * (G * C) + b * C, 16)
                    cps.append(pltpu.async_copy(
                        vals3.at[pl.ds(w0, C), fb], rows.at[b], gsem))
                for cp in cps:
                    cp.wait()
                ops = []
                for b in range(G):
                    ops.append(pltpu.async_copy(
                        rows.at[b], acc.at[didx.at[b]], ssem, add=True))
                for cp in ops:
                    cp.wait()
                return 0

            lax.fori_loop(0, NGRP, group, 0)
            plsc.subcore_barrier()
            for sc_i in range(stripe // subch):
                start = s * subch * (stripe // subch) + sc_i * subch

                @pl.when(start < dc)
                def _():
                    st = pl.multiple_of(start, subch)
                    dst0 = pl.multiple_of(lo + st, 8)
                    pltpu.sync_copy(
                        acc.at[pl.ds(st, subch)],
                        out.at[pl.ds(dst0, subch), fb])
            plsc.subcore_barrier()

    fn = pl.kernel(
        body,
        out_type=jax.ShapeDtypeStruct((out_rows, 4, 16), jnp.float32),
        mesh=plsc.VectorSubcoreMesh(**_MESH),
        compiler_params=pltpu.CompilerParams(use_tc_tiling_on_sc=False),
        scratch_types=[
            pltpu.VMEM((G, C), jnp.int32),
            pltpu.VMEM((G, C), jnp.int32),
            pltpu.VMEM((G, C, 16), jnp.float32),
            pltpu.VMEM((zrows, 16), jnp.float32),
            pltpu.VMEM_SHARED((r_acc, 16), jnp.float32),
            pltpu.SemaphoreType.DMA,
            pltpu.SemaphoreType.DMA,
        ],
    )
    return fn


_seg_edge = None
_seg_node = None


def _get_seg_edge():
    global _seg_edge
    if _seg_edge is None:
        _seg_edge = _make_seg16(
            W=E2, W_pad=E2_PAD, G=16, NGRP=16, C=80,
            rounds=[(r, 0, r, 80000) for r in range(4)],
            dc=80000, r_acc=81920, stripe=5120, subch=320, out_rows=E_PAD)
    return _seg_edge


def _get_seg_node():
    global _seg_node
    if _seg_node is None:
        _seg_node = _make_seg16(
            W=E, W_pad=E_PAD, G=16, NGRP=8, C=80,
            rounds=[(0, 0, 2, 0), (1, 0, 3, 0)],
            dc=N_PAD, r_acc=N_PAD, stripe=640, subch=640, out_rows=N_PAD)
    return _seg_node


_gathers = {}


def _get_gather(V, D, B_pad, G, NGRP, C=128):
    key = (V, D, B_pad, G, NGRP, C)
    if key not in _gathers:
        _gathers[key] = _make_gather(V, D, B_pad, G, NGRP, C)
    return _gathers[key]


def _row_spec(d):
    return pl.BlockSpec((RB, d), lambda i: (i, 0))


def _full_spec(shape):
    nd = len(shape)
    return pl.BlockSpec(shape, lambda i, _n=nd: (0,) * _n)


def _tc_call(body, nblocks, in_specs, out_specs, out_shapes):
    return pl.pallas_call(
        body,
        grid=(nblocks,),
        in_specs=in_specs,
        out_specs=out_specs,
        out_shape=out_shapes,
    )


def _prep_body(hz_ref, rp_ref, w1_ref, w2_ref, at_ref, bt_ref):
    hz = hz_ref[...]
    rp = rp_ref[...]
    z8 = jnp.zeros((RB, 8), jnp.float32)
    a = jnp.dot(hz, w1_ref[...], preferred_element_type=jnp.float32)
    b = jnp.dot(hz, w2_ref[...], preferred_element_type=jnp.float32)
    at_ref[...] = jnp.concatenate([a, rp, z8], axis=1)
    bt_ref[...] = jnp.concatenate([b, rp, z8], axis=1)


def _edge_body(ag_ref, bg_ref, w3_ref, bias_ref, obw0_ref,
               m_ref, ren_ref, o16_ref, rbf8_ref, t0_ref):
    ag = ag_ref[...]
    bg = bg_ref[...]
    ha = ag[:, 0:64]
    hb = bg[:, 0:64]
    o = bg[:, 64:67] - ag[:, 64:67]
    d2 = jnp.sum(o * o, axis=1, keepdims=True) + 1e-12
    d = jnp.sqrt(d2)
    xs = d * (1.0 / CUTOFF)
    inv = 1.0 / xs
    a_c = -(P + 1) * (P + 2) / 2.0
    b_c = float(P * (P + 2))
    c_c = -P * (P + 1) / 2.0
    x4 = (xs * xs) * (xs * xs)
    env = inv + a_c * x4 + b_c * x4 * xs + c_c * x4 * xs * xs
    th = jnp.float32(jnp.pi) * xs
    s_arr = jnp.sin(th)
    c_arr = jnp.cos(th)
    for w in (1, 2, 4, 8, 16):
        sw = s_arr[:, w - 1:w]
        cw = c_arr[:, w - 1:w]
        s_new = jnp.concatenate([s_arr, s_arr * cw + c_arr * sw], axis=1)
        c_new = jnp.concatenate([c_arr, c_arr * cw - s_arr * sw], axis=1)
        s_arr, c_arr = s_new, c_new
    s32 = s_arr[:, 31:32]
    c32 = c_arr[:, 31:32]
    s48 = jnp.concatenate(
        [s_arr, s_arr[:, 0:16] * c32 + c_arr[:, 0:16] * s32], axis=1)
    renv = env * s48
    z13 = jnp.zeros((RB, 13), jnp.float32)
    ren_ref[...] = jnp.concatenate([renv, o, z13], axis=1)
    o16_ref[...] = jnp.concatenate([o, z13], axis=1)
    rbf8 = jnp.concatenate(
        [renv[:, 0:6], jnp.zeros((RB, 2), jnp.float32)], axis=1)
    rbf8_ref[...] = rbf8
    pre = (ha + hb + jnp.dot(rbf8, w3_ref[...],
                             preferred_element_type=jnp.float32)
           + bias_ref[0:1, :])
    m = _silu(pre)
    m_ref[...] = m
    t0_ref[...] = jnp.dot(rbf8, obw0_ref[...],
                          preferred_element_type=jnp.float32) * m


def _line_body(reng_ref, o16g_ref, wcat_ref, sb_ref):
    reng = reng_ref[...]
    re1 = reng[:, 0:48]
    o1 = reng[:, 48:51]
    o2 = o16g_ref[...][:, 0:3]
    dotp = jnp.sum(o1 * o2, axis=1, keepdims=True)
    cx = o1[:, 1:2] * o2[:, 2:3] - o1[:, 2:3] * o2[:, 1:2]
    cy = o1[:, 2:3] * o2[:, 0:1] - o1[:, 0:1] * o2[:, 2:3]
    cz = o1[:, 0:1] * o2[:, 1:2] - o1[:, 1:2] * o2[:, 0:1]
    crn2 = cx * cx + cy * cy + cz * cz + 1e-12
    hyp = jnp.sqrt(dotp * dotp + crn2)
    ca = dotp / hyp
    ts = [jnp.ones((RB, 1), jnp.float32), ca]
    for _ in range(2, NS):
        ts.append(2.0 * ca * ts[-1] - ts[-2])
    parts = [jnp.broadcast_to(t, (RB, NR)) for t in ts]
    parts.append(jnp.zeros((RB, NR), jnp.float32))
    cbf = jnp.concatenate(parts, axis=1)
    sb_ref[...] = jnp.dot(re1 * cbf, wcat_ref[...],
                          preferred_element_type=jnp.float32)


def _c1_body(m_ref, rbf8_ref, wji_ref, wkj_ref, rbfw_ref,
             xji_ref, xkj_ref):
    m = m_ref[...]
    rbf8 = rbf8_ref[...]
    xji_ref[...] = _silu(jnp.dot(m, wji_ref[...],
                                 preferred_element_type=jnp.float32))
    xkj = _silu(jnp.dot(m, wkj_ref[...], preferred_element_type=jnp.float32))
    xkj_ref[...] = xkj * jnp.dot(rbf8, rbfw_ref[...],
                                 preferred_element_type=jnp.float32)


def _c2_body(xk_ref, sb_ref, mt_ref, bil_ref, *, blk):
    xk = xk_ref[...]
    sb = sb_ref[...]
    acc = jnp.zeros((RB, EMB), jnp.float32)
    for j in range(NB):
        acc = acc + jnp.dot(xk, mt_ref[j], preferred_element_type=jnp.float32
                            ) * sb[:, blk * 8 + j:blk * 8 + j + 1]
    bil_ref[...] = acc


def _c3_body(xji_ref, agg_ref, m_ref, rbf8_ref,
             r1_ref, sk_ref, r2_ref, r3_ref, obw_ref,
             mnew_ref, t_ref):
    h = xji_ref[...] + agg_ref[...]
    h = h + _silu(jnp.dot(h, r1_ref[...], preferred_element_type=jnp.float32))
    h = _silu(jnp.dot(h, sk_ref[...],
                      preferred_element_type=jnp.float32)) + m_ref[...]
    h = h + _silu(jnp.dot(h, r2_ref[...], preferred_element_type=jnp.float32))
    h = h + _silu(jnp.dot(h, r3_ref[...], preferred_element_type=jnp.float32))
    mnew_ref[...] = h
    t_ref[...] = jnp.dot(rbf8_ref[...], obw_ref[...],
                         preferred_element_type=jnp.float32) * h


def _out_body(n0_ref, n1_ref, n2_ref, n3_ref, n4_ref,
              dense_ref, outw_ref, p_ref):
    p = jnp.zeros((RB, 16), jnp.float32)
    nrefs = (n0_ref, n1_ref, n2_ref, n3_ref, n4_ref)
    for b in range(NBLK + 1):
        n = nrefs[b][...]
        for j in range(3):
            n = _silu(jnp.dot(n, dense_ref[b, j],
                              preferred_element_type=jnp.float32))
        p = p + jnp.dot(n, outw_ref[b], preferred_element_type=jnp.float32)
    p_ref[...] = p


def _pad_rows(x, rows):
    return jnp.pad(x, ((0, rows - x.shape[0]),) + ((0, 0),) * (x.ndim - 1))


def _pad_idx(ix, n, c=80):
    ix = ix.astype(jnp.int32)
    return jnp.pad(ix, (0, n - ix.shape[0])).reshape(-1, c)


def kernel(Z, R, edge_index, l_edge_index, rbf_freq, sbf_freq, emb_z, W_emb,
           b_emb, ib_Wji, ib_Wkj, ib_rbf_W, ib_sbf_W, ib_bilin, ib_res1,
           ib_skip, ib_res2, ib_res3, ob_rbf_W, ob_dense, ob_out):
    f32 = jnp.float32
    src = edge_index[0]
    dst = edge_index[1]
    l_src = l_edge_index[0]
    l_dst = l_edge_index[1]

    zp = _pad_idx(Z, HZ_PAD, 128)
    srcp = _pad_idx(src, E_PAD)
    dstp = _pad_idx(dst, E_PAD)
    l_srcp = _pad_idx(l_src, E2_PAD)
    l_dstp = _pad_idx(l_dst, E2_PAD)

    hz = _get_gather(95, EMB, HZ_PAD, 8, 1, 128)(emb_z, zp)[:N_PAD]

    rp = _pad_rows(jnp.pad(R, ((0, 0), (0, 5))), N_PAD)
    w1 = W_emb[0:EMB]
    w2 = W_emb[EMB:2 * EMB]
    at, bt = _tc_call(
        _prep_body, N_PAD // RB,
        [_row_spec(EMB), _row_spec(8), _full_spec((EMB, EMB)),
         _full_spec((EMB, EMB))],
        [_row_spec(80), _row_spec(80)],
        [jax.ShapeDtypeStruct((N_PAD, 80), f32)] * 2,
    )(hz, rp, w1, w2)

    gat_e80 = _get_gather(N_PAD, 80, E_PAD, 16, 4, 80)
    ag = gat_e80(at, srcp)
    bg = gat_e80(bt, dstp)

    w3p = jnp.pad(W_emb[2 * EMB:], ((0, 2), (0, 0)))
    biasp = jnp.broadcast_to(b_emb[None, :], (8, EMB))
    obwp = jnp.pad(ob_rbf_W, ((0, 0), (0, 2), (0, 0)))
    m, ren, o16, rbf8, t0 = _tc_call(
        _edge_body, E_PAD // RB,
        [_row_spec(80), _row_spec(80), _full_spec((8, EMB)),
         _full_spec((8, EMB)), _full_spec((8, EMB))],
        [_row_spec(EMB), _row_spec(EMB), _row_spec(16), _row_spec(8),
         _row_spec(EMB)],
        [jax.ShapeDtypeStruct((E_PAD, EMB), f32),
         jax.ShapeDtypeStruct((E_PAD, EMB), f32),
         jax.ShapeDtypeStruct((E_PAD, 16), f32),
         jax.ShapeDtypeStruct((E_PAD, 8), f32),
         jax.ShapeDtypeStruct((E_PAD, EMB), f32)],
    )(ag, bg, w3p, biasp, obwp[0])

    gat_l64 = _get_gather(E_PAD, EMB, E2_PAD, 16, 8, 80)
    gat_l16 = _get_gather(E_PAD, 16, E2_PAD, 16, 8, 80)
    reng = gat_l64(ren, l_srcp)
    o16g = gat_l16(o16, l_dstp)

    wcat = jnp.pad(
        jnp.transpose(ib_sbf_W, (1, 0, 2)).reshape(NSR, NBLK * NB),
        ((0, 48 - NSR), (0, 0)))
    (sb,) = _tc_call(
        _line_body, E2_PAD // RB,
        [_row_spec(EMB), _row_spec(16), _full_spec((48, NBLK * NB))],
        [_row_spec(NBLK * NB)],
        [jax.ShapeDtypeStruct((E2_PAD, NBLK * NB), f32)],
    )(reng, o16g, wcat)

    seg_edge = _get_seg_edge()
    seg_node = _get_seg_node()

    nodes = [seg_node(t0.reshape(E_PAD, 4, 16), dstp).reshape(N_PAD, EMB)]
    rbfwp = jnp.pad(ib_rbf_W, ((0, 0), (0, 2), (0, 0)))
    for i in range(NBLK):
        xji, xkj = _tc_call(
            _c1_body, E_PAD // RB,
            [_row_spec(EMB), _row_spec(8), _full_spec((EMB, EMB)),
             _full_spec((EMB, EMB)), _full_spec((8, EMB))],
            [_row_spec(EMB), _row_spec(EMB)],
            [jax.ShapeDtypeStruct((E_PAD, EMB), f32)] * 2,
        )(m, rbf8, ib_Wji[i], ib_Wkj[i], rbfwp[i])
        xkg = gat_l64(xkj, l_srcp)
        mt = jnp.transpose(ib_bilin[i], (1, 2, 0))
        (bil,) = _tc_call(
            functools.partial(_c2_body, blk=i), E2_PAD // RB,
            [_row_spec(EMB), _row_spec(NBLK * NB),
             _full_spec((NB, EMB, EMB))],
            [_row_spec(EMB)],
            [jax.ShapeDtypeStruct((E2_PAD, EMB), f32)],
        )(xkg, sb, mt)
        agg = seg_edge(bil.reshape(E2_PAD, 4, 16), l_dstp).reshape(E_PAD, EMB)
        m, t = _tc_call(
            _c3_body, E_PAD // RB,
            [_row_spec(EMB), _row_spec(EMB), _row_spec(EMB), _row_spec(8),
             _full_spec((EMB, EMB)), _full_spec((EMB, EMB)),
             _full_spec((EMB, EMB)), _full_spec((EMB, EMB)),
             _full_spec((8, EMB))],
            [_row_spec(EMB), _row_spec(EMB)],
            [jax.ShapeDtypeStruct((E_PAD, EMB), f32)] * 2,
        )(xji, agg, m, rbf8, ib_res1[i], ib_skip[i], ib_res2[i], ib_res3[i],
          obwp[i + 1])
        nodes.append(seg_node(t.reshape(E_PAD, 4, 16), dstp).reshape(N_PAD, EMB))

    outwp = jnp.pad(ob_out, ((0, 0), (0, 0), (0, 16 - NT)))
    (pout,) = _tc_call(
        _out_body, N_PAD // RB,
        [_row_spec(EMB)] * 5 + [_full_spec((NBLK + 1, 3, EMB, EMB)),
                                _full_spec((NBLK + 1, EMB, 16))],
        [_row_spec(16)],
        [jax.ShapeDtypeStruct((N_PAD, 16), f32)],
    )(*nodes, ob_dense, outwp)
    return pout[:N, :NT]


# speedup vs baseline: 1.2987x; 1.1740x over previous
import functools

import jax
import jax.numpy as jnp
from jax import lax
from jax.experimental import pallas as pl
from jax.experimental.pallas import tpu as pltpu
from jax.experimental.pallas import tpu_sc as plsc

N = 10000
E = 160000
E2 = 320000
EMB = 64
NR = 6
NS = 7
NSR = NS * NR
NB = 8
NBLK = 4
NT = 12
CUTOFF = 5.0
P = 5

NC = 2
NSUB = 16
NWORK = NC * NSUB
LANES = 16

N_PAD = 10240
HZ_PAD = 32768
E_PAD = 163840
E2_PAD = 327680

RB = 640

_MESH = dict(core_axis_name="c", subcore_axis_name="s", num_cores=NC,
             num_subcores=NSUB)


def _silu(x):
    return x / (1.0 + jnp.exp(-x))


def _make_gather(V, D, B_pad, G, NGRP, C=128):
    bt = B_pad // NWORK
    assert bt == G * NGRP * C

    def body(table, idx2, out, ichunk, rows, gsem, osem):
        c = lax.axis_index("c")
        s = lax.axis_index("s")
        wid = s * NC + c
        base = wid * bt

        def group(g, _):
            row0 = pl.multiple_of((base + g * (G * C)) // C, 8)
            pltpu.sync_copy(idx2.at[pl.ds(row0, G)], ichunk)
            cps = []
            for b in range(G):
                cps.append(pltpu.async_copy(
                    table.at[ichunk.at[b]], rows.at[b], gsem))
            for cp in cps:
                cp.wait()
            ops = []
            for b in range(G):
                off = pl.multiple_of(base + g * (G * C) + b * C, 16)
                ops.append(pltpu.async_copy(
                    rows.at[b], out.at[pl.ds(off, C)], osem))
            for cp in ops:
                cp.wait()
            return 0

        lax.fori_loop(0, NGRP, group, 0)

    fn = pl.kernel(
        body,
        out_type=jax.ShapeDtypeStruct((B_pad, D), jnp.float32),
        mesh=plsc.VectorSubcoreMesh(**_MESH),
        compiler_params=pltpu.CompilerParams(use_tc_tiling_on_sc=False),
        scratch_types=[
            pltpu.VMEM((G, C), jnp.int32),
            pltpu.VMEM((G, C, D), jnp.float32),
            pltpu.SemaphoreType.DMA,
            pltpu.SemaphoreType.DMA,
        ],
    )
    return fn


def _make_seg16(W, W_pad, G, NGRP, rounds, dc, r_acc, stripe, subch, out_rows,
                C=128, nvals=1):
    wt = W_pad // NSUB
    assert wt == G * NGRP * C
    assert stripe * NSUB == r_acc
    zrows = stripe // 4 if stripe % 4 == 0 else stripe
    nz = stripe // zrows
    trash = r_acc - 8

    def body(*refs):
        vals = refs[:nvals]
        idx2 = refs[nvals]
        outs = refs[nvals + 1:2 * nvals + 1]
        ichunk, didx, rows, zbuf, acc, gsem, ssem = refs[2 * nvals + 1:]
        c = lax.axis_index("c")
        s = lax.axis_index("s")

        zv = jnp.zeros((LANES,), jnp.float32)

        def zb(i, _):
            zbuf[i, :] = zv
            return 0
        lax.fori_loop(0, zrows, zb, 0)

        for v in range(nvals):
            for (fb0, lo0, fb1, lo1) in rounds:
                fb = jnp.where(c == 0, fb0, fb1)
                lo = jnp.where(c == 0, lo0, lo1)
                for z in range(nz):
                    pltpu.sync_copy(zbuf,
                                    acc.at[pl.ds(s * stripe + z * zrows,
                                                 zrows)])
                plsc.subcore_barrier()
                base = s * wt

                def group(g, _):
                    row0 = pl.multiple_of((base + g * (G * C)) // C, 8)
                    pltpu.sync_copy(idx2.at[pl.ds(row0, G)], ichunk)
                    for b in range(G):
                        for k in range(C // 16):
                            iv = ichunk[b, pl.ds(k * 16, 16)]
                            pos = (base + g * (G * C) + b * C + k * 16
                                   + lax.iota(jnp.int32, 16))
                            ok = ((pos < W) & (iv >= lo) & (iv < lo + dc))
                            didx[b, pl.ds(k * 16, 16)] = jnp.where(
                                ok, iv - lo, trash)
                    cps = []
                    for b in range(G):
                        w0 = pl.multiple_of(base + g * (G * C) + b * C, 16)
                        cps.append(pltpu.async_copy(
                            vals[v].at[fb, pl.ds(w0, C)], rows.at[b], gsem))
                    for cp in cps:
                        cp.wait()
                    ops = []
                    for b in range(G):
                        ops.append(pltpu.async_copy(
                            rows.at[b], acc.at[didx.at[b]], ssem, add=True))
                    for cp in ops:
                        cp.wait()
                    return 0

                lax.fori_loop(0, NGRP, group, 0)
                plsc.subcore_barrier()
                for sc_i in range(stripe // subch):
                    start = s * subch * (stripe // subch) + sc_i * subch

                    @pl.when(start < dc)
                    def _():
                        st = pl.multiple_of(start, subch)
                        dst0 = pl.multiple_of(lo + st, 8)
                        fb16 = pl.multiple_of(fb * 16, 16)
                        pltpu.sync_copy(
                            acc.at[pl.ds(st, subch)],
                            outs[v].at[pl.ds(dst0, subch), pl.ds(fb16, 16)])
                plsc.subcore_barrier()

    fn = pl.kernel(
        body,
        out_type=[jax.ShapeDtypeStruct((out_rows, EMB), jnp.float32)] * nvals,
        mesh=plsc.VectorSubcoreMesh(**_MESH),
        compiler_params=pltpu.CompilerParams(use_tc_tiling_on_sc=False),
        scratch_types=[
            pltpu.VMEM((G, C), jnp.int32),
            pltpu.VMEM((G, C), jnp.int32),
            pltpu.VMEM((G, C, 16), jnp.float32),
            pltpu.VMEM((zrows, 16), jnp.float32),
            pltpu.VMEM_SHARED((r_acc, 16), jnp.float32),
            pltpu.SemaphoreType.DMA,
            pltpu.SemaphoreType.DMA,
        ],
    )
    return fn


_seg_edge = None
_seg_node = None


def _get_seg_edge():
    global _seg_edge
    if _seg_edge is None:
        _seg_edge = _make_seg16(
            W=E2, W_pad=E2_PAD, G=16, NGRP=16, C=80,
            rounds=[(r, 0, r, 80000) for r in range(4)],
            dc=80000, r_acc=81920, stripe=5120, subch=320, out_rows=E_PAD)
    return _seg_edge


def _get_seg_node():
    global _seg_node
    if _seg_node is None:
        _seg_node = _make_seg16(
            W=E, W_pad=E_PAD, G=16, NGRP=8, C=80,
            rounds=[(0, 0, 2, 0), (1, 0, 3, 0)],
            dc=N_PAD, r_acc=N_PAD, stripe=640, subch=640, out_rows=N_PAD,
            nvals=NBLK + 1)
    return _seg_node


_gathers = {}


def _get_gather(V, D, B_pad, G, NGRP, C=128):
    key = (V, D, B_pad, G, NGRP, C)
    if key not in _gathers:
        _gathers[key] = _make_gather(V, D, B_pad, G, NGRP, C)
    return _gathers[key]


def _row_spec(d):
    return pl.BlockSpec((RB, d), lambda i: (i, 0))


def _fb_spec():
    return pl.BlockSpec((4, RB, 16), lambda i: (0, i, 0))


def _full_spec(shape):
    nd = len(shape)
    return pl.BlockSpec(shape, lambda i, _n=nd: (0,) * _n)


def _tc_call(body, nblocks, in_specs, out_specs, out_shapes):
    return pl.pallas_call(
        body,
        grid=(nblocks,),
        in_specs=in_specs,
        out_specs=out_specs,
        out_shape=out_shapes,
    )


def _prep_body(hz_ref, rp_ref, w1_ref, w2_ref, at_ref, bt_ref):
    hz = hz_ref[...]
    rp = rp_ref[...]
    z8 = jnp.zeros((RB, 8), jnp.float32)
    a = jnp.dot(hz, w1_ref[...], preferred_element_type=jnp.float32)
    b = jnp.dot(hz, w2_ref[...], preferred_element_type=jnp.float32)
    at_ref[...] = jnp.concatenate([a, rp, z8], axis=1)
    bt_ref[...] = jnp.concatenate([b, rp, z8], axis=1)


def _edge_body(ag_ref, bg_ref, w3_ref, bias_ref, obw0_ref,
               m_ref, ren_ref, o16_ref, rbf8_ref, t0_ref):
    ag = ag_ref[...]
    bg = bg_ref[...]
    ha = ag[:, 0:64]
    hb = bg[:, 0:64]
    o = bg[:, 64:67] - ag[:, 64:67]
    d2 = jnp.sum(o * o, axis=1, keepdims=True) + 1e-12
    d = jnp.sqrt(d2)
    xs = d * (1.0 / CUTOFF)
    inv = 1.0 / xs
    a_c = -(P + 1) * (P + 2) / 2.0
    b_c = float(P * (P + 2))
    c_c = -P * (P + 1) / 2.0
    x4 = (xs * xs) * (xs * xs)
    env = inv + a_c * x4 + b_c * x4 * xs + c_c * x4 * xs * xs
    th = jnp.float32(jnp.pi) * xs
    s_arr = jnp.sin(th)
    c_arr = jnp.cos(th)
    for w in (1, 2, 4, 8, 16):
        sw = s_arr[:, w - 1:w]
        cw = c_arr[:, w - 1:w]
        s_new = jnp.concatenate([s_arr, s_arr * cw + c_arr * sw], axis=1)
        c_new = jnp.concatenate([c_arr, c_arr * cw - s_arr * sw], axis=1)
        s_arr, c_arr = s_new, c_new
    s32 = s_arr[:, 31:32]
    c32 = c_arr[:, 31:32]
    s48 = jnp.concatenate(
        [s_arr, s_arr[:, 0:16] * c32 + c_arr[:, 0:16] * s32], axis=1)
    renv = env * s48
    z13 = jnp.zeros((RB, 13), jnp.float32)
    ren_ref[...] = jnp.concatenate([renv, o, z13], axis=1)
    o16_ref[...] = jnp.concatenate([o, z13], axis=1)
    rbf8 = jnp.concatenate(
        [renv[:, 0:6], jnp.zeros((RB, 2), jnp.float32)], axis=1)
    rbf8_ref[...] = rbf8
    pre = (ha + hb + jnp.dot(rbf8, w3_ref[...],
                             preferred_element_type=jnp.float32)
           + bias_ref[0:1, :])
    m = _silu(pre)
    m_ref[...] = m
    t0 = jnp.dot(rbf8, obw0_ref[...],
                 preferred_element_type=jnp.float32) * m
    for f in range(4):
        t0_ref[f] = t0[:, f * 16:(f + 1) * 16]


def _line_body(reng_ref, o16g_ref, wcat_ref, sb_ref):
    reng = reng_ref[...]
    re1 = reng[:, 0:48]
    o1 = reng[:, 48:51]
    o2 = o16g_ref[...][:, 0:3]
    dotp = jnp.sum(o1 * o2, axis=1, keepdims=True)
    cx = o1[:, 1:2] * o2[:, 2:3] - o1[:, 2:3] * o2[:, 1:2]
    cy = o1[:, 2:3] * o2[:, 0:1] - o1[:, 0:1] * o2[:, 2:3]
    cz = o1[:, 0:1] * o2[:, 1:2] - o1[:, 1:2] * o2[:, 0:1]
    crn2 = cx * cx + cy * cy + cz * cz + 1e-12
    hyp = jnp.sqrt(dotp * dotp + crn2)
    ca = dotp / hyp
    ts = [jnp.ones((RB, 1), jnp.float32), ca]
    for _ in range(2, NS):
        ts.append(2.0 * ca * ts[-1] - ts[-2])
    parts = [jnp.broadcast_to(t, (RB, NR)) for t in ts]
    parts.append(jnp.zeros((RB, NR), jnp.float32))
    cbf = jnp.concatenate(parts, axis=1)
    sb_ref[...] = jnp.dot(re1 * cbf, wcat_ref[...],
                          preferred_element_type=jnp.float32)


def _c1_body(m_ref, rbf8_ref, wji_ref, wkj_ref, rbfw_ref,
             xji_ref, xkj_ref):
    m = m_ref[...]
    rbf8 = rbf8_ref[...]
    xji_ref[...] = _silu(jnp.dot(m, wji_ref[...],
                                 preferred_element_type=jnp.float32))
    xkj = _silu(jnp.dot(m, wkj_ref[...], preferred_element_type=jnp.float32))
    xkj_ref[...] = xkj * jnp.dot(rbf8, rbfw_ref[...],
                                 preferred_element_type=jnp.float32)


def _c2_body(xk_ref, sb_ref, mt_ref, bil_ref, *, blk):
    xk = xk_ref[...]
    sb = sb_ref[...]
    acc = jnp.zeros((RB, EMB), jnp.float32)
    for j in range(NB):
        acc = acc + jnp.dot(xk, mt_ref[j], preferred_element_type=jnp.float32
                            ) * sb[:, blk * 8 + j:blk * 8 + j + 1]
    for f in range(4):
        bil_ref[f] = acc[:, f * 16:(f + 1) * 16]


def _c3_body(xji_ref, agg_ref, m_ref, rbf8_ref,
             r1_ref, sk_ref, r2_ref, r3_ref, obw_ref,
             mnew_ref, t_ref):
    h = xji_ref[...] + agg_ref[...]
    h = h + _silu(jnp.dot(h, r1_ref[...], preferred_element_type=jnp.float32))
    h = _silu(jnp.dot(h, sk_ref[...],
                      preferred_element_type=jnp.float32)) + m_ref[...]
    h = h + _silu(jnp.dot(h, r2_ref[...], preferred_element_type=jnp.float32))
    h = h + _silu(jnp.dot(h, r3_ref[...], preferred_element_type=jnp.float32))
    mnew_ref[...] = h
    t = jnp.dot(rbf8_ref[...], obw_ref[...],
                preferred_element_type=jnp.float32) * h
    for f in range(4):
        t_ref[f] = t[:, f * 16:(f + 1) * 16]


def _out_body(n0_ref, n1_ref, n2_ref, n3_ref, n4_ref,
              dense_ref, outw_ref, p_ref):
    p = jnp.zeros((RB, 16), jnp.float32)
    nrefs = (n0_ref, n1_ref, n2_ref, n3_ref, n4_ref)
    for b in range(NBLK + 1):
        n = nrefs[b][...]
        for j in range(3):
            n = _silu(jnp.dot(n, dense_ref[b, j],
                              preferred_element_type=jnp.float32))
        p = p + jnp.dot(n, outw_ref[b], preferred_element_type=jnp.float32)
    p_ref[...] = p


def _pad_rows(x, rows):
    return jnp.pad(x, ((0, rows - x.shape[0]),) + ((0, 0),) * (x.ndim - 1))


def _pad_idx(ix, n, c=80):
    ix = ix.astype(jnp.int32)
    return jnp.pad(ix, (0, n - ix.shape[0])).reshape(-1, c)


def kernel(Z, R, edge_index, l_edge_index, rbf_freq, sbf_freq, emb_z, W_emb,
           b_emb, ib_Wji, ib_Wkj, ib_rbf_W, ib_sbf_W, ib_bilin, ib_res1,
           ib_skip, ib_res2, ib_res3, ob_rbf_W, ob_dense, ob_out):
    f32 = jnp.float32
    src = edge_index[0]
    dst = edge_index[1]
    l_src = l_edge_index[0]
    l_dst = l_edge_index[1]

    zp = _pad_idx(Z, HZ_PAD, 128)
    srcp = _pad_idx(src, E_PAD)
    dstp = _pad_idx(dst, E_PAD)
    l_srcp = _pad_idx(l_src, E2_PAD)
    l_dstp = _pad_idx(l_dst, E2_PAD)

    hz = _get_gather(95, EMB, HZ_PAD, 8, 1, 128)(emb_z, zp)[:N_PAD]

    rp = _pad_rows(jnp.pad(R, ((0, 0), (0, 5))), N_PAD)
    w1 = W_emb[0:EMB]
    w2 = W_emb[EMB:2 * EMB]
    at, bt = _tc_call(
        _prep_body, N_PAD // RB,
        [_row_spec(EMB), _row_spec(8), _full_spec((EMB, EMB)),
         _full_spec((EMB, EMB))],
        [_row_spec(80), _row_spec(80)],
        [jax.ShapeDtypeStruct((N_PAD, 80), f32)] * 2,
    )(hz, rp, w1, w2)

    gat_e80 = _get_gather(N_PAD, 80, E_PAD, 16, 4, 80)
    ag = gat_e80(at, srcp)
    bg = gat_e80(bt, dstp)

    w3p = jnp.pad(W_emb[2 * EMB:], ((0, 2), (0, 0)))
    biasp = jnp.broadcast_to(b_emb[None, :], (8, EMB))
    obwp = jnp.pad(ob_rbf_W, ((0, 0), (0, 2), (0, 0)))
    m, ren, o16, rbf8, t0 = _tc_call(
        _edge_body, E_PAD // RB,
        [_row_spec(80), _row_spec(80), _full_spec((8, EMB)),
         _full_spec((8, EMB)), _full_spec((8, EMB))],
        [_row_spec(EMB), _row_spec(EMB), _row_spec(16), _row_spec(8),
         _fb_spec()],
        [jax.ShapeDtypeStruct((E_PAD, EMB), f32),
         jax.ShapeDtypeStruct((E_PAD, EMB), f32),
         jax.ShapeDtypeStruct((E_PAD, 16), f32),
         jax.ShapeDtypeStruct((E_PAD, 8), f32),
         jax.ShapeDtypeStruct((4, E_PAD, 16), f32)],
    )(ag, bg, w3p, biasp, obwp[0])

    gat_l64 = _get_gather(E_PAD, EMB, E2_PAD, 16, 8, 80)
    gat_l16 = _get_gather(E_PAD, 16, E2_PAD, 16, 8, 80)
    reng = gat_l64(ren, l_srcp)
    o16g = gat_l16(o16, l_dstp)

    wcat = jnp.pad(
        jnp.transpose(ib_sbf_W, (1, 0, 2)).reshape(NSR, NBLK * NB),
        ((0, 48 - NSR), (0, 0)))
    (sb,) = _tc_call(
        _line_body, E2_PAD // RB,
        [_row_spec(EMB), _row_spec(16), _full_spec((48, NBLK * NB))],
        [_row_spec(NBLK * NB)],
        [jax.ShapeDtypeStruct((E2_PAD, NBLK * NB), f32)],
    )(reng, o16g, wcat)

    seg_edge = _get_seg_edge()
    seg_node = _get_seg_node()

    tlist = [t0]
    rbfwp = jnp.pad(ib_rbf_W, ((0, 0), (0, 2), (0, 0)))
    for i in range(NBLK):
        xji, xkj = _tc_call(
            _c1_body, E_PAD // RB,
            [_row_spec(EMB), _row_spec(8), _full_spec((EMB, EMB)),
             _full_spec((EMB, EMB)), _full_spec((8, EMB))],
            [_row_spec(EMB), _row_spec(EMB)],
            [jax.ShapeDtypeStruct((E_PAD, EMB), f32)] * 2,
        )(m, rbf8, ib_Wji[i], ib_Wkj[i], rbfwp[i])
        xkg = gat_l64(xkj, l_srcp)
        mt = jnp.transpose(ib_bilin[i], (1, 2, 0))
        (bil,) = _tc_call(
            functools.partial(_c2_body, blk=i), E2_PAD // RB,
            [_row_spec(EMB), _row_spec(NBLK * NB),
             _full_spec((NB, EMB, EMB))],
            [_fb_spec()],
            [jax.ShapeDtypeStruct((4, E2_PAD, 16), f32)],
        )(xkg, sb, mt)
        (agg,) = seg_edge(bil, l_dstp)
        m, t = _tc_call(
            _c3_body, E_PAD // RB,
            [_row_spec(EMB), _row_spec(EMB), _row_spec(EMB), _row_spec(8),
             _full_spec((EMB, EMB)), _full_spec((EMB, EMB)),
             _full_spec((EMB, EMB)), _full_spec((EMB, EMB)),
             _full_spec((8, EMB))],
            [_row_spec(EMB), _fb_spec()],
            [jax.ShapeDtypeStruct((E_PAD, EMB), f32),
             jax.ShapeDtypeStruct((4, E_PAD, 16), f32)],
        )(xji, agg, m, rbf8, ib_res1[i], ib_skip[i], ib_res2[i], ib_res3[i],
          obwp[i + 1])
        tlist.append(t)

    nodes = seg_node(*tlist, dstp)

    outwp = jnp.pad(ob_out, ((0, 0), (0, 0), (0, 16 - NT)))
    (pout,) = _tc_call(
        _out_body, N_PAD // RB,
        [_row_spec(EMB)] * 5 + [_full_spec((NBLK + 1, 3, EMB, EMB)),
                                _full_spec((NBLK + 1, EMB, 16))],
        [_row_spec(16)],
        [jax.ShapeDtypeStruct((N_PAD, 16), f32)],
    )(*nodes, ob_dense, outwp)
    return pout[:N, :NT]


# speedup vs baseline: 1.3073x; 1.0066x over previous
import functools

import jax
import jax.numpy as jnp
from jax import lax
from jax.experimental import pallas as pl
from jax.experimental.pallas import tpu as pltpu
from jax.experimental.pallas import tpu_sc as plsc

N = 10000
E = 160000
E2 = 320000
EMB = 64
NR = 6
NS = 7
NSR = NS * NR
NB = 8
NBLK = 4
NT = 12
CUTOFF = 5.0
P = 5

NC = 2
NSUB = 16
NWORK = NC * NSUB
LANES = 16

N_PAD = 10240
HZ_PAD = 32768
E_PAD = 163840
E2_PAD = 327680

RB = 640

_MESH = dict(core_axis_name="c", subcore_axis_name="s", num_cores=NC,
             num_subcores=NSUB)


def _silu(x):
    return x / (1.0 + jnp.exp(-x))


def _make_gather(specs, B_pad, G, NGRP, C=128):
    bt = B_pad // NWORK
    assert bt == G * NGRP * C
    nsp = len(specs)
    buf_keys = []
    for (_, d, dt) in specs:
        if (d, dt) not in buf_keys:
            buf_keys.append((d, dt))

    def body(*refs):
        tables = refs[0:2 * nsp:2]
        idxs = refs[1:2 * nsp:2]
        outs = refs[2 * nsp:3 * nsp]
        ichunk = refs[3 * nsp]
        bufs = refs[3 * nsp + 1:3 * nsp + 1 + len(buf_keys)]
        gsem, osem = refs[3 * nsp + 1 + len(buf_keys):]
        c = lax.axis_index("c")
        s = lax.axis_index("s")
        wid = s * NC + c
        base = wid * bt

        for p, (_, d, dt) in enumerate(specs):
            rows = bufs[buf_keys.index((d, dt))]
            table = tables[p]
            idx2 = idxs[p]
            out = outs[p]

            def group(g, _):
                row0 = pl.multiple_of((base + g * (G * C)) // C, 8)
                pltpu.sync_copy(idx2.at[pl.ds(row0, G)], ichunk)
                cps = []
                for b in range(G):
                    cps.append(pltpu.async_copy(
                        table.at[ichunk.at[b]], rows.at[b], gsem))
                for cp in cps:
                    cp.wait()
                ops = []
                for b in range(G):
                    off = pl.multiple_of(base + g * (G * C) + b * C, 16)
                    ops.append(pltpu.async_copy(
                        rows.at[b], out.at[pl.ds(off, C)], osem))
                for cp in ops:
                    cp.wait()
                return 0

            lax.fori_loop(0, NGRP, group, 0)

    fn = pl.kernel(
        body,
        out_type=[jax.ShapeDtypeStruct((B_pad, d), dt)
                  for (_, d, dt) in specs],
        mesh=plsc.VectorSubcoreMesh(**_MESH),
        compiler_params=pltpu.CompilerParams(use_tc_tiling_on_sc=False),
        scratch_types=(
            [pltpu.VMEM((G, C), jnp.int32)]
            + [pltpu.VMEM((G, C, d), dt) for (d, dt) in buf_keys]
            + [pltpu.SemaphoreType.DMA, pltpu.SemaphoreType.DMA]),
    )
    return fn


def _make_seg16(W, W_pad, G, NGRP, rounds, dc, r_acc, stripe, subch, out_rows,
                C=128, nvals=1):
    wt = W_pad // NSUB
    assert wt == G * NGRP * C
    assert stripe * NSUB == r_acc
    zrows = stripe // 4 if stripe % 4 == 0 else stripe
    nz = stripe // zrows
    trash = r_acc - 8

    def body(*refs):
        vals = refs[:nvals]
        idx2 = refs[nvals]
        outs = refs[nvals + 1:2 * nvals + 1]
        ichunk, didx, rows, zbuf, acc, gsem, ssem = refs[2 * nvals + 1:]
        c = lax.axis_index("c")
        s = lax.axis_index("s")

        zv = jnp.zeros((LANES,), jnp.float32)

        def zb(i, _):
            zbuf[i, :] = zv
            return 0
        lax.fori_loop(0, zrows, zb, 0)

        for v in range(nvals):
            for (fb0, lo0, fb1, lo1) in rounds:
                fb = jnp.where(c == 0, fb0, fb1)
                lo = jnp.where(c == 0, lo0, lo1)
                for z in range(nz):
                    pltpu.sync_copy(zbuf,
                                    acc.at[pl.ds(s * stripe + z * zrows,
                                                 zrows)])
                plsc.subcore_barrier()
                base = s * wt

                def group(g, _):
                    row0 = pl.multiple_of((base + g * (G * C)) // C, 8)
                    pltpu.sync_copy(idx2.at[pl.ds(row0, G)], ichunk)
                    for b in range(G):
                        for k in range(C // 16):
                            iv = ichunk[b, pl.ds(k * 16, 16)]
                            pos = (base + g * (G * C) + b * C + k * 16
                                   + lax.iota(jnp.int32, 16))
                            ok = ((pos < W) & (iv >= lo) & (iv < lo + dc))
                            didx[b, pl.ds(k * 16, 16)] = jnp.where(
                                ok, iv - lo, trash)
                    cps = []
                    for b in range(G):
                        w0 = pl.multiple_of(base + g * (G * C) + b * C, 16)
                        cps.append(pltpu.async_copy(
                            vals[v].at[fb, pl.ds(w0, C)], rows.at[b], gsem))
                    for cp in cps:
                        cp.wait()
                    ops = []
                    for b in range(G):
                        ops.append(pltpu.async_copy(
                            rows.at[b], acc.at[didx.at[b]], ssem, add=True))
                    for cp in ops:
                        cp.wait()
                    return 0

                lax.fori_loop(0, NGRP, group, 0)
                plsc.subcore_barrier()
                for sc_i in range(stripe // subch):
                    start = s * subch * (stripe // subch) + sc_i * subch

                    @pl.when(start < dc)
                    def _():
                        st = pl.multiple_of(start, subch)
                        dst0 = pl.multiple_of(lo + st, 8)
                        fb16 = pl.multiple_of(fb * 16, 16)
                        pltpu.sync_copy(
                            acc.at[pl.ds(st, subch)],
                            outs[v].at[pl.ds(dst0, subch), pl.ds(fb16, 16)])
                plsc.subcore_barrier()

    fn = pl.kernel(
        body,
        out_type=[jax.ShapeDtypeStruct((out_rows, EMB), jnp.float32)] * nvals,
        mesh=plsc.VectorSubcoreMesh(**_MESH),
        compiler_params=pltpu.CompilerParams(use_tc_tiling_on_sc=False),
        scratch_types=[
            pltpu.VMEM((G, C), jnp.int32),
            pltpu.VMEM((G, C), jnp.int32),
            pltpu.VMEM((G, C, 16), jnp.float32),
            pltpu.VMEM((zrows, 16), jnp.float32),
            pltpu.VMEM_SHARED((r_acc, 16), jnp.float32),
            pltpu.SemaphoreType.DMA,
            pltpu.SemaphoreType.DMA,
        ],
    )
    return fn


_seg_edge = None
_seg_node = None


def _get_seg_edge():
    global _seg_edge
    if _seg_edge is None:
        _seg_edge = _make_seg16(
            W=E2, W_pad=E2_PAD, G=16, NGRP=16, C=80,
            rounds=[(r, 0, r, 80000) for r in range(4)],
            dc=80000, r_acc=81920, stripe=5120, subch=320, out_rows=E_PAD)
    return _seg_edge


def _get_seg_node():
    global _seg_node
    if _seg_node is None:
        _seg_node = _make_seg16(
            W=E, W_pad=E_PAD, G=16, NGRP=8, C=80,
            rounds=[(0, 0, 2, 0), (1, 0, 3, 0)],
            dc=N_PAD, r_acc=N_PAD, stripe=640, subch=640, out_rows=N_PAD,
            nvals=NBLK + 1)
    return _seg_node


_gathers = {}


def _get_gather(specs, B_pad, G, NGRP, C=128):
    key = (tuple(specs), B_pad, G, NGRP, C)
    if key not in _gathers:
        _gathers[key] = _make_gather(list(specs), B_pad, G, NGRP, C)
    return _gathers[key]


def _row_spec(d):
    return pl.BlockSpec((RB, d), lambda i: (i, 0))


def _fb_spec():
    return pl.BlockSpec((4, RB, 16), lambda i: (0, i, 0))


def _full_spec(shape):
    nd = len(shape)
    return pl.BlockSpec(shape, lambda i, _n=nd: (0,) * _n)


def _tc_call(body, nblocks, in_specs, out_specs, out_shapes):
    return pl.pallas_call(
        body,
        grid=(nblocks,),
        in_specs=in_specs,
        out_specs=out_specs,
        out_shape=out_shapes,
    )


def _prep_body(hz_ref, rp_ref, w1_ref, w2_ref, at_ref, bt_ref):
    hz = hz_ref[...]
    rp = rp_ref[...]
    z8 = jnp.zeros((RB, 8), jnp.float32)
    a = jnp.dot(hz, w1_ref[...], preferred_element_type=jnp.float32)
    b = jnp.dot(hz, w2_ref[...], preferred_element_type=jnp.float32)
    at_ref[...] = jnp.concatenate([a, rp, z8], axis=1)
    bt_ref[...] = jnp.concatenate([b, rp, z8], axis=1)


def _edge_body(ag_ref, bg_ref, w3_ref, bias_ref, obw0_ref,
               m_ref, ren_ref, o16_ref, rbf8_ref, t0_ref):
    ag = ag_ref[...]
    bg = bg_ref[...]
    ha = ag[:, 0:64]
    hb = bg[:, 0:64]
    o = bg[:, 64:67] - ag[:, 64:67]
    d2 = jnp.sum(o * o, axis=1, keepdims=True) + 1e-12
    d = jnp.sqrt(d2)
    xs = d * (1.0 / CUTOFF)
    inv = 1.0 / xs
    a_c = -(P + 1) * (P + 2) / 2.0
    b_c = float(P * (P + 2))
    c_c = -P * (P + 1) / 2.0
    x4 = (xs * xs) * (xs * xs)
    env = inv + a_c * x4 + b_c * x4 * xs + c_c * x4 * xs * xs
    th = jnp.float32(jnp.pi) * xs
    s_arr = jnp.sin(th)
    c_arr = jnp.cos(th)
    for w in (1, 2, 4, 8, 16):
        sw = s_arr[:, w - 1:w]
        cw = c_arr[:, w - 1:w]
        s_new = jnp.concatenate([s_arr, s_arr * cw + c_arr * sw], axis=1)
        c_new = jnp.concatenate([c_arr, c_arr * cw - s_arr * sw], axis=1)
        s_arr, c_arr = s_new, c_new
    s32 = s_arr[:, 31:32]
    c32 = c_arr[:, 31:32]
    s48 = jnp.concatenate(
        [s_arr, s_arr[:, 0:16] * c32 + c_arr[:, 0:16] * s32], axis=1)
    renv = env * s48
    z13 = jnp.zeros((RB, 13), jnp.float32)
    ren_ref[...] = jnp.concatenate([renv, o, z13], axis=1)
    o16_ref[...] = jnp.concatenate([o, z13], axis=1)
    rbf8 = jnp.concatenate(
        [renv[:, 0:6], jnp.zeros((RB, 2), jnp.float32)], axis=1)
    rbf8_ref[...] = rbf8
    pre = (ha + hb + jnp.dot(rbf8, w3_ref[...],
                             preferred_element_type=jnp.float32)
           + bias_ref[0:1, :])
    m = _silu(pre)
    m_ref[...] = m
    t0 = jnp.dot(rbf8, obw0_ref[...],
                 preferred_element_type=jnp.float32) * m
    for f in range(4):
        t0_ref[f] = t0[:, f * 16:(f + 1) * 16]


def _line_body(reng_ref, o16g_ref, wcat_ref, sb_ref):
    reng = reng_ref[...]
    re1 = reng[:, 0:48]
    o1 = reng[:, 48:51]
    o2 = o16g_ref[...][:, 0:3]
    dotp = jnp.sum(o1 * o2, axis=1, keepdims=True)
    cx = o1[:, 1:2] * o2[:, 2:3] - o1[:, 2:3] * o2[:, 1:2]
    cy = o1[:, 2:3] * o2[:, 0:1] - o1[:, 0:1] * o2[:, 2:3]
    cz = o1[:, 0:1] * o2[:, 1:2] - o1[:, 1:2] * o2[:, 0:1]
    crn2 = cx * cx + cy * cy + cz * cz + 1e-12
    hyp = jnp.sqrt(dotp * dotp + crn2)
    ca = dotp / hyp
    ts = [jnp.ones((RB, 1), jnp.float32), ca]
    for _ in range(2, NS):
        ts.append(2.0 * ca * ts[-1] - ts[-2])
    parts = [jnp.broadcast_to(t, (RB, NR)) for t in ts]
    parts.append(jnp.zeros((RB, NR), jnp.float32))
    cbf = jnp.concatenate(parts, axis=1)
    sb_ref[...] = jnp.dot(re1 * cbf, wcat_ref[...],
                          preferred_element_type=jnp.float32)


def _c1_body(m_ref, rbf8_ref, wji_ref, wkj_ref, rbfw_ref,
             xji_ref, xkj_ref):
    m = m_ref[...]
    rbf8 = rbf8_ref[...]
    xji_ref[...] = _silu(jnp.dot(m, wji_ref[...],
                                 preferred_element_type=jnp.float32))
    xkj = _silu(jnp.dot(m, wkj_ref[...], preferred_element_type=jnp.float32))
    xkj_ref[...] = (xkj * jnp.dot(rbf8, rbfw_ref[...],
                                  preferred_element_type=jnp.float32)
                    ).astype(jnp.bfloat16)


def _c2_body(xk_ref, sb_ref, mt_ref, bil_ref, *, blk):
    xk = xk_ref[...].astype(jnp.float32)
    sb = sb_ref[...]
    acc = jnp.zeros((RB, EMB), jnp.float32)
    for j in range(NB):
        acc = acc + jnp.dot(xk, mt_ref[j], preferred_element_type=jnp.float32
                            ) * sb[:, blk * 8 + j:blk * 8 + j + 1]
    for f in range(4):
        bil_ref[f] = acc[:, f * 16:(f + 1) * 16]


def _c3_body(xji_ref, agg_ref, m_ref, rbf8_ref,
             r1_ref, sk_ref, r2_ref, r3_ref, obw_ref,
             mnew_ref, t_ref):
    h = xji_ref[...] + agg_ref[...]
    h = h + _silu(jnp.dot(h, r1_ref[...], preferred_element_type=jnp.float32))
    h = _silu(jnp.dot(h, sk_ref[...],
                      preferred_element_type=jnp.float32)) + m_ref[...]
    h = h + _silu(jnp.dot(h, r2_ref[...], preferred_element_type=jnp.float32))
    h = h + _silu(jnp.dot(h, r3_ref[...], preferred_element_type=jnp.float32))
    mnew_ref[...] = h
    t = jnp.dot(rbf8_ref[...], obw_ref[...],
                preferred_element_type=jnp.float32) * h
    for f in range(4):
        t_ref[f] = t[:, f * 16:(f + 1) * 16]


def _out_body(n0_ref, n1_ref, n2_ref, n3_ref, n4_ref,
              dense_ref, outw_ref, p_ref):
    p = jnp.zeros((RB, 16), jnp.float32)
    nrefs = (n0_ref, n1_ref, n2_ref, n3_ref, n4_ref)
    for b in range(NBLK + 1):
        n = nrefs[b][...]
        for j in range(3):
            n = _silu(jnp.dot(n, dense_ref[b, j],
                              preferred_element_type=jnp.float32))
        p = p + jnp.dot(n, outw_ref[b], preferred_element_type=jnp.float32)
    p_ref[...] = p


def _pad_rows(x, rows):
    return jnp.pad(x, ((0, rows - x.shape[0]),) + ((0, 0),) * (x.ndim - 1))


def _pad_idx(ix, n, c=80):
    ix = ix.astype(jnp.int32)
    return jnp.pad(ix, (0, n - ix.shape[0])).reshape(-1, c)


def kernel(Z, R, edge_index, l_edge_index, rbf_freq, sbf_freq, emb_z, W_emb,
           b_emb, ib_Wji, ib_Wkj, ib_rbf_W, ib_sbf_W, ib_bilin, ib_res1,
           ib_skip, ib_res2, ib_res3, ob_rbf_W, ob_dense, ob_out):
    f32 = jnp.float32
    src = edge_index[0]
    dst = edge_index[1]
    l_src = l_edge_index[0]
    l_dst = l_edge_index[1]

    zp = _pad_idx(Z, HZ_PAD, 128)
    srcp = _pad_idx(src, E_PAD)
    dstp = _pad_idx(dst, E_PAD)
    l_srcp = _pad_idx(l_src, E2_PAD)
    l_dstp = _pad_idx(l_dst, E2_PAD)

    (hz,) = _get_gather(((95, EMB, jnp.float32),), HZ_PAD, 8, 1, 128)(
        emb_z, zp)
    hz = hz[:N_PAD]

    rp = _pad_rows(jnp.pad(R, ((0, 0), (0, 5))), N_PAD)
    w1 = W_emb[0:EMB]
    w2 = W_emb[EMB:2 * EMB]
    at, bt = _tc_call(
        _prep_body, N_PAD // RB,
        [_row_spec(EMB), _row_spec(8), _full_spec((EMB, EMB)),
         _full_spec((EMB, EMB))],
        [_row_spec(80), _row_spec(80)],
        [jax.ShapeDtypeStruct((N_PAD, 80), f32)] * 2,
    )(hz, rp, w1, w2)

    ag, bg = _get_gather(((N_PAD, 80, jnp.float32),) * 2, E_PAD, 16, 4, 80)(
        at, srcp, bt, dstp)

    w3p = jnp.pad(W_emb[2 * EMB:], ((0, 2), (0, 0)))
    biasp = jnp.broadcast_to(b_emb[None, :], (8, EMB))
    obwp = jnp.pad(ob_rbf_W, ((0, 0), (0, 2), (0, 0)))
    m, ren, o16, rbf8, t0 = _tc_call(
        _edge_body, E_PAD // RB,
        [_row_spec(80), _row_spec(80), _full_spec((8, EMB)),
         _full_spec((8, EMB)), _full_spec((8, EMB))],
        [_row_spec(EMB), _row_spec(EMB), _row_spec(16), _row_spec(8),
         _fb_spec()],
        [jax.ShapeDtypeStruct((E_PAD, EMB), f32),
         jax.ShapeDtypeStruct((E_PAD, EMB), f32),
         jax.ShapeDtypeStruct((E_PAD, 16), f32),
         jax.ShapeDtypeStruct((E_PAD, 8), f32),
         jax.ShapeDtypeStruct((4, E_PAD, 16), f32)],
    )(ag, bg, w3p, biasp, obwp[0])

    reng, o16g = _get_gather(
        ((E_PAD, EMB, jnp.float32), (E_PAD, 16, jnp.float32)),
        E2_PAD, 16, 8, 80)(ren, l_srcp, o16, l_dstp)
    gat_xk = _get_gather(((E_PAD, EMB, jnp.bfloat16),), E2_PAD, 16, 8, 80)

    wcat = jnp.pad(
        jnp.transpose(ib_sbf_W, (1, 0, 2)).reshape(NSR, NBLK * NB),
        ((0, 48 - NSR), (0, 0)))
    (sb,) = _tc_call(
        _line_body, E2_PAD // RB,
        [_row_spec(EMB), _row_spec(16), _full_spec((48, NBLK * NB))],
        [_row_spec(NBLK * NB)],
        [jax.ShapeDtypeStruct((E2_PAD, NBLK * NB), f32)],
    )(reng, o16g, wcat)

    seg_edge = _get_seg_edge()
    seg_node = _get_seg_node()

    tlist = [t0]
    rbfwp = jnp.pad(ib_rbf_W, ((0, 0), (0, 2), (0, 0)))
    for i in range(NBLK):
        xji, xkj = _tc_call(
            _c1_body, E_PAD // RB,
            [_row_spec(EMB), _row_spec(8), _full_spec((EMB, EMB)),
             _full_spec((EMB, EMB)), _full_spec((8, EMB))],
            [_row_spec(EMB), _row_spec(EMB)],
            [jax.ShapeDtypeStruct((E_PAD, EMB), f32),
             jax.ShapeDtypeStruct((E_PAD, EMB), jnp.bfloat16)],
        )(m, rbf8, ib_Wji[i], ib_Wkj[i], rbfwp[i])
        (xkg,) = gat_xk(xkj, l_srcp)
        mt = jnp.transpose(ib_bilin[i], (1, 2, 0))
        (bil,) = _tc_call(
            functools.partial(_c2_body, blk=i), E2_PAD // RB,
            [_row_spec(EMB), _row_spec(NBLK * NB),
             _full_spec((NB, EMB, EMB))],
            [_fb_spec()],
            [jax.ShapeDtypeStruct((4, E2_PAD, 16), f32)],
        )(xkg, sb, mt)
        (agg,) = seg_edge(bil, l_dstp)
        m, t = _tc_call(
            _c3_body, E_PAD // RB,
            [_row_spec(EMB), _row_spec(EMB), _row_spec(EMB), _row_spec(8),
             _full_spec((EMB, EMB)), _full_spec((EMB, EMB)),
             _full_spec((EMB, EMB)), _full_spec((EMB, EMB)),
             _full_spec((8, EMB))],
            [_row_spec(EMB), _fb_spec()],
            [jax.ShapeDtypeStruct((E_PAD, EMB), f32),
             jax.ShapeDtypeStruct((4, E_PAD, 16), f32)],
        )(xji, agg, m, rbf8, ib_res1[i], ib_skip[i], ib_res2[i], ib_res3[i],
          obwp[i + 1])
        tlist.append(t)

    nodes = seg_node(*tlist, dstp)

    outwp = jnp.pad(ob_out, ((0, 0), (0, 0), (0, 16 - NT)))
    (pout,) = _tc_call(
        _out_body, N_PAD // RB,
        [_row_spec(EMB)] * 5 + [_full_spec((NBLK + 1, 3, EMB, EMB)),
                                _full_spec((NBLK + 1, EMB, 16))],
        [_row_spec(16)],
        [jax.ShapeDtypeStruct((N_PAD, 16), f32)],
    )(*nodes, ob_dense, outwp)
    return pout[:N, :NT]
